# baseline (speedup 1.0000x reference)
import functools

import jax
import jax.numpy as jnp
from jax import lax
from jax.experimental import pallas as pl
from jax.experimental.pallas import tpu as pltpu
from jax.experimental.pallas import tpu_sc as plsc

_LANES = 16
_CH = 16
_NBUF = 4


def _sc_add(x, t, B, S, D):
    info = plsc.get_sparse_core_info()
    NC, NS = info.num_cores, info.num_subcores
    NW = NC * NS
    s_per_w = S // NW
    nch = s_per_w // _CH
    nsteps = nch * B

    mesh = plsc.VectorSubcoreMesh(core_axis_name="c", subcore_axis_name="s")

    @functools.partial(
        pl.kernel,
        mesh=mesh,
        out_type=jax.ShapeDtypeStruct((B, S, D), jnp.float32),
        scratch_types=(
            [pltpu.VMEM((_CH, D), jnp.float32) for _ in range(_NBUF)]
            + [pltpu.VMEM((_CH, D), jnp.float32) for _ in range(2)]
            + [pltpu.SemaphoreType.DMA for _ in range(2 * _NBUF + 2)]
        ),
    )
    def k(x_hbm, t_hbm, out_hbm, *bufs_and_sems):
        o_bufs = bufs_and_sems[:_NBUF]
        t_bufs = bufs_and_sems[_NBUF:_NBUF + 2]
        sx = bufs_and_sems[_NBUF + 2:2 * _NBUF + 2]
        so = bufs_and_sems[2 * _NBUF + 2:3 * _NBUF + 2]
        st = bufs_and_sems[3 * _NBUF + 2:]
        wid = lax.axis_index("s") * NC + lax.axis_index("c")
        s0 = wid * s_per_w

        def x_copy(step, b):
            i, bat = step // B, step % B
            return pltpu.make_async_copy(
                x_hbm.at[bat, pl.ds(s0 + i * _CH, _CH), :], o_bufs[b], sx[b])

        def t_copy(i, b):
            return pltpu.make_async_copy(
                t_hbm.at[pl.ds(s0 + i * _CH, _CH), :], t_bufs[b], st[b])

        def out_copy(step, b):
            i, bat = step // B, step % B
            return pltpu.make_async_copy(
                o_bufs[b], out_hbm.at[bat, pl.ds(s0 + i * _CH, _CH), :], so[b])

        for p in range(min(_NBUF - 1, nsteps)):
            x_copy(p, p % _NBUF).start()
        t_copy(0, 0).start()
        if nch > 1:
            t_copy(1, 1).start()
        for step in range(nsteps):
            b = step % _NBUF
            i = step // B
            pf = step + _NBUF - 1
            if pf < nsteps:
                if step >= 1:
                    out_copy(pf - _NBUF, pf % _NBUF).wait()
                x_copy(pf, pf % _NBUF).start()
            x_copy(step, b).wait()
            if step % B == 0:
                t_copy(i, i % 2).wait()
            o_v, t_v = o_bufs[b], t_bufs[i % 2]

            H = _CH // 4
            ii, bat = step // B, step % B
            for h in range(4):

                @plsc.parallel_loop(h * H * D, (h + 1) * H * D, _LANES,
                                    unroll=8)
                def add_body(off):
                    r = off // D
                    c = off % D
                    plsc.addupdate(o_v.at[r, pl.ds(c, _LANES)],
                                   t_v[r, pl.ds(c, _LANES)])

                pltpu.make_async_copy(
                    o_bufs[b].at[pl.ds(h * H, H), :],
                    out_hbm.at[bat, pl.ds(s0 + ii * _CH + h * H, H), :],
                    so[b]).start()
            if step % B == B - 1 and i + 2 < nch:
                t_copy(i + 2, i % 2).start()
        for tail in range(max(nsteps - _NBUF + 1, 0), nsteps):
            out_copy(tail, tail % _NBUF).wait()

    return k(x, t)


def kernel(x, embeddings_table):
    B, S, D = x.shape
    return _sc_add(x, embeddings_table, B, S, D)


# speedup vs baseline: 1.1039x; 1.1039x over previous
import functools

import jax
import jax.numpy as jnp
from jax import lax
from jax.experimental import pallas as pl
from jax.experimental.pallas import tpu as pltpu
from jax.experimental.pallas import tpu_sc as plsc

_LANES = 16
_CH = 16
_NBUF = 4


def _sc_add(x, t, B, S, D):
    info = plsc.get_sparse_core_info()
    NC, NS = info.num_cores, info.num_subcores
    NW = NC * NS
    s_per_w = S // NW
    nch = s_per_w // _CH
    nsteps = nch * B

    mesh = plsc.VectorSubcoreMesh(core_axis_name="c", subcore_axis_name="s")

    @functools.partial(
        pl.kernel,
        mesh=mesh,
        out_type=jax.ShapeDtypeStruct((B, S, D), jnp.float32),
        scratch_types=(
            [pltpu.VMEM((_CH, D), jnp.float32) for _ in range(_NBUF)]
            + [pltpu.VMEM((_CH, D), jnp.float32) for _ in range(2)]
            + [pltpu.SemaphoreType.DMA for _ in range(2 * _NBUF + 2)]
        ),
    )
    def k(x_hbm, t_hbm, out_hbm, *bufs_and_sems):
        o_bufs = bufs_and_sems[:_NBUF]
        t_bufs = bufs_and_sems[_NBUF:_NBUF + 2]
        sx = bufs_and_sems[_NBUF + 2:2 * _NBUF + 2]
        so = bufs_and_sems[2 * _NBUF + 2:3 * _NBUF + 2]
        st = bufs_and_sems[3 * _NBUF + 2:]
        wid = lax.axis_index("s") * NC + lax.axis_index("c")
        s0 = wid * s_per_w

        def x_copy(step, b):
            i, bat = step // B, step % B
            return pltpu.make_async_copy(
                x_hbm.at[bat, pl.ds(s0 + i * _CH, _CH), :], o_bufs[b], sx[b])

        def t_copy(i, b):
            return pltpu.make_async_copy(
                t_hbm.at[pl.ds(s0 + i * _CH, _CH), :], t_bufs[b], st[b])

        def out_copy(step, b):
            i, bat = step // B, step % B
            return pltpu.make_async_copy(
                o_bufs[b], out_hbm.at[bat, pl.ds(s0 + i * _CH, _CH), :], so[b])

        for p in range(min(_NBUF - 1, nsteps)):
            x_copy(p, p % _NBUF).start()
        t_copy(0, 0).start()
        if nch > 1:
            t_copy(1, 1).start()
        for step in range(nsteps):
            b = step % _NBUF
            i = step // B
            pf = step + _NBUF - 1
            if pf < nsteps:
                if step >= 1:
                    out_copy(pf - _NBUF, pf % _NBUF).wait()
                x_copy(pf, pf % _NBUF).start()
            x_copy(step, b).wait()
            if step % B == 0:
                t_copy(i, i % 2).wait()
            o_v, t_v = o_bufs[b], t_bufs[i % 2]

            H = _CH // 2
            ii, bat = step // B, step % B
            for h in range(2):

                @plsc.parallel_loop(h * H * D, (h + 1) * H * D, _LANES,
                                    unroll=8)
                def add_body(off):
                    r = off // D
                    c = off % D
                    plsc.addupdate(o_v.at[r, pl.ds(c, _LANES)],
                                   t_v[r, pl.ds(c, _LANES)])

                pltpu.make_async_copy(
                    o_bufs[b].at[pl.ds(h * H, H), :],
                    out_hbm.at[bat, pl.ds(s0 + ii * _CH + h * H, H), :],
                    so[b]).start()
            if step % B == B - 1 and i + 2 < nch:
                t_copy(i + 2, i % 2).start()
        for tail in range(max(nsteps - _NBUF + 1, 0), nsteps):
            out_copy(tail, tail % _NBUF).wait()

    return k(x, t)


def kernel(x, embeddings_table):
    B, S, D = x.shape
    return _sc_add(x, embeddings_table, B, S, D)
